# Initial kernel scaffold; baseline (speedup 1.0000x reference)
import jax
import jax.numpy as jnp
from jax.experimental import pallas as pl


def kernel(means, quats, features, flame_vertices, canonical_vertices, me_W1, me_b1, me_W2, me_b2, pe_W1, pe_b1, pe_W2, pe_b2, md_W1, md_b1, md_W2, md_b2, md_W3, md_b3, ld_W1, ld_b1, ld_W2, ld_b2, ld_W3, ld_b3):
    raise NotImplementedError("write your pallas kernel here")


# speedup vs baseline: 8.7479x; 8.7479x over previous
import functools

import jax
import jax.numpy as jnp
from jax import lax
from jax.experimental import pallas as pl
from jax.experimental.pallas import tpu as pltpu
from jax.experimental.pallas import tpu_sc as plsc

N_GAUSS = 50000
N_VERTS = 5143
WINDOW = 8
LATENT = 32
HID = 32

NQP = 50176
NVP = 5248
NFLAT = 3 * NQP
NW = 32
PER_W = 5120
NFLAT_PAD = NW * PER_W
CHUNK = 128
N_CHUNKS = PER_W // CHUNK

_BIG = 1e30


def _knn_body(m_ref, c_ref, idx_ref):
    d2 = jnp.dot(m_ref[...], c_ref[...], preferred_element_type=jnp.float32)
    rows = d2.shape[0]
    col = lax.broadcasted_iota(jnp.int32, d2.shape, 1)
    picks = []
    for _ in range(3):
        mn = jnp.min(d2, axis=1, keepdims=True)
        eq = d2 == mn
        pick = jnp.min(jnp.where(eq, col, jnp.int32(2**30)), axis=1, keepdims=True)
        picks.append(pick)
        d2 = jnp.where(col == pick, jnp.float32(_BIG), d2)
    lane = lax.broadcasted_iota(jnp.int32, (rows, 8), 1)
    out = jnp.where(lane == 0, picks[0],
                    jnp.where(lane == 1, picks[1],
                              jnp.where(lane == 2, picks[2], 0)))
    idx_ref[...] = out


def _knn(means8, cvt8):
    blk = 128
    return pl.pallas_call(
        _knn_body,
        grid=(NQP // blk,),
        in_specs=[
            pl.BlockSpec((blk, 8), lambda i: (i, 0)),
            pl.BlockSpec((8, NVP), lambda i: (0, 0)),
        ],
        out_specs=pl.BlockSpec((blk, 8), lambda i: (i, 0)),
        out_shape=jax.ShapeDtypeStruct((NQP, 8), jnp.int32),
    )(means8, cvt8)


def _gather_body(table_hbm, idx_hbm, out_hbm, idx_v, rows_v, sem):
    wid = lax.axis_index("s") * 2 + lax.axis_index("c")
    base = wid * PER_W
    pltpu.sync_copy(idx_hbm.at[pl.ds(wid * N_CHUNKS, N_CHUNKS)], idx_v)

    def chunk(c, carry):
        pltpu.async_copy(table_hbm.at[idx_v.at[c]], rows_v, sem).wait()
        pltpu.sync_copy(rows_v, out_hbm.at[pl.ds(base + c * CHUNK, CHUNK)])
        return carry

    lax.fori_loop(0, N_CHUNKS, chunk, 0)


def _gather_sc(table, flat_idx2d):
    mesh = plsc.VectorSubcoreMesh(core_axis_name="c", subcore_axis_name="s")
    run = functools.partial(
        pl.kernel,
        out_type=jax.ShapeDtypeStruct((NFLAT_PAD, 32), jnp.float32),
        mesh=mesh,
        scratch_types=[
            pltpu.VMEM((N_CHUNKS, CHUNK), jnp.int32),
            pltpu.VMEM((CHUNK, 32), jnp.float32),
            pltpu.SemaphoreType.DMA,
        ],
        compiler_params=pltpu.CompilerParams(use_tc_tiling_on_sc=False),
    )(_gather_body)
    return run(table, flat_idx2d)


def _silu(x):
    return x / (1.0 + jnp.exp(-x))


def _dense_body(mq_ref, g_ref,
                meW1_ref, meb1_ref, meW2_ref, meb2_ref,
                peW1_ref, peb1_ref, peW2_ref, peb2_ref,
                mdW1a_ref, mdW1b_ref, mdb1_ref, mdW2_ref, mdb2_ref,
                mdW3_ref, mdb3_ref,
                ldW1a_ref, ldW1b_ref, ldb1_ref, ldW2_ref, ldb2_ref,
                ldW3_ref, ldb3_ref,
                out_ref):
    mq = mq_ref[...]
    g0 = g_ref[0]
    g1 = g_ref[1]
    g2 = g_ref[2]
    rows = mq.shape[0]

    means3 = mq[:, 0:3]
    v0 = g0[:, 24:27]
    e1 = g1[:, 24:27] - v0
    e2 = g2[:, 24:27] - v0
    ep = means3 - v0
    d00 = jnp.sum(e1 * e1, axis=1, keepdims=True)
    d01 = jnp.sum(e1 * e2, axis=1, keepdims=True)
    d11 = jnp.sum(e2 * e2, axis=1, keepdims=True)
    d20 = jnp.sum(ep * e1, axis=1, keepdims=True)
    d21 = jnp.sum(ep * e2, axis=1, keepdims=True)
    denom = d00 * d11 - d01 * d01 + 1e-8
    v = (d11 * d20 - d01 * d21) / denom
    w = (d00 * d21 - d01 * d20) / denom
    u = 1.0 - v - w

    nm = u * g0 + v * g1 + w * g2

    def mm(a, w_ref):
        return jnp.dot(a, w_ref[...], preferred_element_type=jnp.float32)

    x = _silu(mm(_silu(mm(nm, meW1_ref) + meb1_ref[...]), meW2_ref) + meb2_ref[...])
    pe = _silu(mm(_silu(mm(mq, peW1_ref) + peb1_ref[...]), peW2_ref) + peb2_ref[...])

    h = _silu(mm(x, mdW1a_ref) + mm(pe, mdW1b_ref) + mdb1_ref[...])
    h = _silu(mm(h, mdW2_ref) + mdb2_ref[...])
    mv = mm(h, mdW3_ref) + mdb3_ref[...]

    lane = lax.broadcasted_iota(jnp.int32, (rows, 8), 1)
    scale = jnp.where(lane < 3, jnp.float32(0.001),
                      jnp.where(lane < 7, jnp.float32(0.01), jnp.float32(0.0)))
    upd8 = mq + scale * mv

    l = _silu(mm(x, ldW1a_ref) + mm(pe, ldW1b_ref) + ldb1_ref[...])
    l = _silu(mm(l, ldW2_ref) + ldb2_ref[...])
    feat = mm(l, ldW3_ref) + ldb3_ref[...]

    out_ref[...] = jnp.concatenate(
        [upd8, feat, jnp.zeros((rows, 8), jnp.float32)], axis=1)


def _dense(mq, g3, weights):
    blk = 256
    w_specs = [pl.BlockSpec(w.shape, lambda i: tuple(0 for _ in w.shape))
               for w in weights]
    return pl.pallas_call(
        _dense_body,
        grid=(NQP // blk,),
        in_specs=[
            pl.BlockSpec((blk, 8), lambda i: (i, 0)),
            pl.BlockSpec((3, blk, 32), lambda i: (0, i, 0)),
        ] + w_specs,
        out_specs=pl.BlockSpec((blk, 48), lambda i: (i, 0)),
        out_shape=jax.ShapeDtypeStruct((NQP, 48), jnp.float32),
    )(mq, g3, *weights)


def _pad_rows(a, n):
    return jnp.concatenate(
        [a, jnp.zeros((n - a.shape[0],) + a.shape[1:], a.dtype)], axis=0)


def kernel(means, quats, features, flame_vertices, canonical_vertices,
           me_W1, me_b1, me_W2, me_b2, pe_W1, pe_b1, pe_W2, pe_b2,
           md_W1, md_b1, md_W2, md_b2, md_W3, md_b3,
           ld_W1, ld_b1, ld_W2, ld_b2, ld_W3, ld_b3):
    f32 = jnp.float32

    meansp = _pad_rows(means, NQP)
    means8 = jnp.concatenate(
        [meansp, jnp.zeros((NQP, 4), f32), jnp.ones((NQP, 1), f32)], axis=1)
    cn2 = jnp.sum(canonical_vertices * canonical_vertices, axis=1)
    top = jnp.concatenate(
        [-2.0 * canonical_vertices.T, jnp.zeros((4, N_VERTS), f32), cn2[None]],
        axis=0)
    padcols = jnp.concatenate(
        [jnp.zeros((7, NVP - N_VERTS), f32),
         jnp.full((1, NVP - N_VERTS), _BIG, f32)], axis=0)
    cvt8 = jnp.concatenate([top, padcols], axis=1)

    idx8 = _knn(means8, cvt8)

    flat_idx = idx8[:, :3].T.reshape(NFLAT)
    flat_idx = jnp.concatenate(
        [flat_idx, jnp.zeros((NFLAT_PAD - NFLAT,), jnp.int32)])
    flat_idx2d = flat_idx.reshape(NFLAT_PAD // CHUNK, CHUNK)

    vm = jnp.transpose(flame_vertices, (1, 0, 2)).reshape(N_VERTS, WINDOW * 3)
    table = jnp.concatenate(
        [vm, canonical_vertices, jnp.zeros((N_VERTS, 5), f32)], axis=1)
    table = _pad_rows(table, NVP)

    gathered = _gather_sc(table, flat_idx2d)
    g3 = gathered[:NFLAT].reshape(3, NQP, 32)

    quatsp = _pad_rows(quats, NQP)
    mq = jnp.concatenate([meansp, quatsp, jnp.zeros((NQP, 1), f32)], axis=1)

    z8 = jnp.zeros((8, HID), f32)
    meW1p = jnp.concatenate([me_W1, z8], axis=0)
    peW1p = jnp.concatenate([pe_W1, jnp.zeros((1, HID), f32)], axis=0)
    mdW3p = jnp.concatenate([md_W3, jnp.zeros((HID, 1), f32)], axis=1)
    mdb3p = jnp.concatenate([md_b3, jnp.zeros((1,), f32)])[None]
    weights = [
        meW1p, me_b1[None], me_W2, me_b2[None],
        peW1p, pe_b1[None], pe_W2, pe_b2[None],
        md_W1[:HID], md_W1[HID:], md_b1[None], md_W2, md_b2[None],
        mdW3p, mdb3p,
        ld_W1[:HID], ld_W1[HID:], ld_b1[None], ld_W2, ld_b2[None],
        ld_W3, ld_b3[None],
    ]
    out48 = _dense(mq, g3, weights)

    new_means = out48[:N_GAUSS, 0:3]
    new_quats = out48[:N_GAUSS, 3:7]
    new_features = out48[:N_GAUSS, 8:8 + LATENT]
    return (new_means, new_quats, new_features, jnp.float32(0.0))


# speedup vs baseline: 9.5437x; 1.0910x over previous
import functools

import jax
import jax.numpy as jnp
from jax import lax
from jax.experimental import pallas as pl
from jax.experimental.pallas import tpu as pltpu
from jax.experimental.pallas import tpu_sc as plsc

N_GAUSS = 50000
N_VERTS = 5143
WINDOW = 8
LATENT = 32
HID = 32

NQP = 50176
NVP = 5248
NFLAT = 3 * NQP
NW = 32
PER_W = 5120
NFLAT_PAD = NW * PER_W
CHUNK = 128
N_CHUNKS = PER_W // CHUNK

_BIG = 1e30


def _knn_body(m_ref, c_ref, idx_ref):
    d2 = jnp.dot(m_ref[...], c_ref[...], preferred_element_type=jnp.float32)
    rows = d2.shape[0]
    col = lax.broadcasted_iota(jnp.int32, d2.shape, 1)
    picks = []
    for _ in range(3):
        mn = jnp.min(d2, axis=1, keepdims=True)
        eq = d2 == mn
        pick = jnp.min(jnp.where(eq, col, jnp.int32(2**30)), axis=1, keepdims=True)
        picks.append(pick)
        d2 = jnp.where(col == pick, jnp.float32(_BIG), d2)
    lane = lax.broadcasted_iota(jnp.int32, (rows, 8), 1)
    out = jnp.where(lane == 0, picks[0],
                    jnp.where(lane == 1, picks[1],
                              jnp.where(lane == 2, picks[2], 0)))
    idx_ref[...] = out


def _knn(means8, cvt8):
    blk = 256
    return pl.pallas_call(
        _knn_body,
        grid=(NQP // blk,),
        in_specs=[
            pl.BlockSpec((blk, 8), lambda i: (i, 0)),
            pl.BlockSpec((8, NVP), lambda i: (0, 0)),
        ],
        out_specs=pl.BlockSpec((blk, 8), lambda i: (i, 0)),
        out_shape=jax.ShapeDtypeStruct((NQP, 8), jnp.int32),
    )(means8, cvt8)


def _gather_body(table_hbm, idx_hbm, out_hbm, idx_v, rows_a, rows_b, sem_a, sem_b):
    wid = lax.axis_index("s") * 2 + lax.axis_index("c")
    base = wid * PER_W
    pltpu.sync_copy(idx_hbm.at[pl.ds(wid * N_CHUNKS, N_CHUNKS)], idx_v)

    def pair(p, carry):
        c0 = 2 * p
        c1 = c0 + 1
        cp0 = pltpu.async_copy(table_hbm.at[idx_v.at[c0]], rows_a, sem_a)
        cp1 = pltpu.async_copy(table_hbm.at[idx_v.at[c1]], rows_b, sem_b)
        cp0.wait()
        pltpu.sync_copy(rows_a, out_hbm.at[pl.ds(base + c0 * CHUNK, CHUNK)])
        cp1.wait()
        pltpu.sync_copy(rows_b, out_hbm.at[pl.ds(base + c1 * CHUNK, CHUNK)])
        return carry

    lax.fori_loop(0, N_CHUNKS // 2, pair, 0)


def _gather_sc(table, flat_idx2d):
    mesh = plsc.VectorSubcoreMesh(core_axis_name="c", subcore_axis_name="s")
    run = functools.partial(
        pl.kernel,
        out_type=jax.ShapeDtypeStruct((NFLAT_PAD, 32), jnp.float32),
        mesh=mesh,
        scratch_types=[
            pltpu.VMEM((N_CHUNKS, CHUNK), jnp.int32),
            pltpu.VMEM((CHUNK, 32), jnp.float32),
            pltpu.VMEM((CHUNK, 32), jnp.float32),
            pltpu.SemaphoreType.DMA,
            pltpu.SemaphoreType.DMA,
        ],
        compiler_params=pltpu.CompilerParams(use_tc_tiling_on_sc=False),
    )(_gather_body)
    return run(table, flat_idx2d)


def _silu(x):
    return x / (1.0 + jnp.exp(-x))


def _dense_body(mq_ref, g_ref,
                meW1_ref, meb1_ref, meW2_ref, meb2_ref,
                peW1_ref, peb1_ref, peW2_ref, peb2_ref,
                mdW1a_ref, mdW1b_ref, mdb1_ref, mdW2_ref, mdb2_ref,
                mdW3_ref, mdb3_ref,
                ldW1a_ref, ldW1b_ref, ldb1_ref, ldW2_ref, ldb2_ref,
                ldW3_ref, ldb3_ref,
                out_ref):
    mq = mq_ref[...]
    g0 = g_ref[0]
    g1 = g_ref[1]
    g2 = g_ref[2]
    rows = mq.shape[0]

    means3 = mq[:, 0:3]
    v0 = g0[:, 24:27]
    e1 = g1[:, 24:27] - v0
    e2 = g2[:, 24:27] - v0
    ep = means3 - v0
    d00 = jnp.sum(e1 * e1, axis=1, keepdims=True)
    d01 = jnp.sum(e1 * e2, axis=1, keepdims=True)
    d11 = jnp.sum(e2 * e2, axis=1, keepdims=True)
    d20 = jnp.sum(ep * e1, axis=1, keepdims=True)
    d21 = jnp.sum(ep * e2, axis=1, keepdims=True)
    denom = d00 * d11 - d01 * d01 + 1e-8
    v = (d11 * d20 - d01 * d21) / denom
    w = (d00 * d21 - d01 * d20) / denom
    u = 1.0 - v - w

    nm = u * g0 + v * g1 + w * g2

    def mm(a, w_ref):
        return jnp.dot(a, w_ref[...], preferred_element_type=jnp.float32)

    x = _silu(mm(_silu(mm(nm, meW1_ref) + meb1_ref[...]), meW2_ref) + meb2_ref[...])
    pe = _silu(mm(_silu(mm(mq, peW1_ref) + peb1_ref[...]), peW2_ref) + peb2_ref[...])

    h = _silu(mm(x, mdW1a_ref) + mm(pe, mdW1b_ref) + mdb1_ref[...])
    h = _silu(mm(h, mdW2_ref) + mdb2_ref[...])
    mv = mm(h, mdW3_ref) + mdb3_ref[...]

    lane = lax.broadcasted_iota(jnp.int32, (rows, 8), 1)
    scale = jnp.where(lane < 3, jnp.float32(0.001),
                      jnp.where(lane < 7, jnp.float32(0.01), jnp.float32(0.0)))
    upd8 = mq + scale * mv

    l = _silu(mm(x, ldW1a_ref) + mm(pe, ldW1b_ref) + ldb1_ref[...])
    l = _silu(mm(l, ldW2_ref) + ldb2_ref[...])
    feat = mm(l, ldW3_ref) + ldb3_ref[...]

    out_ref[...] = jnp.concatenate(
        [upd8, feat, jnp.zeros((rows, 8), jnp.float32)], axis=1)


def _dense(mq, g3, weights):
    blk = 256
    w_specs = [pl.BlockSpec(w.shape, lambda i: tuple(0 for _ in w.shape))
               for w in weights]
    return pl.pallas_call(
        _dense_body,
        grid=(NQP // blk,),
        in_specs=[
            pl.BlockSpec((blk, 8), lambda i: (i, 0)),
            pl.BlockSpec((3, blk, 32), lambda i: (0, i, 0)),
        ] + w_specs,
        out_specs=pl.BlockSpec((blk, 48), lambda i: (i, 0)),
        out_shape=jax.ShapeDtypeStruct((NQP, 48), jnp.float32),
    )(mq, g3, *weights)


def _pad_rows(a, n):
    return jnp.concatenate(
        [a, jnp.zeros((n - a.shape[0],) + a.shape[1:], a.dtype)], axis=0)


def kernel(means, quats, features, flame_vertices, canonical_vertices,
           me_W1, me_b1, me_W2, me_b2, pe_W1, pe_b1, pe_W2, pe_b2,
           md_W1, md_b1, md_W2, md_b2, md_W3, md_b3,
           ld_W1, ld_b1, ld_W2, ld_b2, ld_W3, ld_b3):
    f32 = jnp.float32

    m8q = jnp.concatenate(
        [_pad_rows(means, NQP), _pad_rows(quats, NQP), jnp.ones((NQP, 1), f32)],
        axis=1)
    cn2 = jnp.sum(canonical_vertices * canonical_vertices, axis=1)
    top = jnp.concatenate(
        [-2.0 * canonical_vertices.T, jnp.zeros((4, N_VERTS), f32), cn2[None]],
        axis=0)
    padcols = jnp.concatenate(
        [jnp.zeros((7, NVP - N_VERTS), f32),
         jnp.full((1, NVP - N_VERTS), _BIG, f32)], axis=0)
    cvt8 = jnp.concatenate([top, padcols], axis=1)

    idx8 = _knn(m8q, cvt8)

    flat_idx = idx8[:, :3].T.reshape(NFLAT)
    flat_idx = jnp.concatenate(
        [flat_idx, jnp.zeros((NFLAT_PAD - NFLAT,), jnp.int32)])
    flat_idx2d = flat_idx.reshape(NFLAT_PAD // CHUNK, CHUNK)

    vm = jnp.transpose(flame_vertices, (1, 0, 2)).reshape(N_VERTS, WINDOW * 3)
    table = jnp.concatenate(
        [vm, canonical_vertices, jnp.zeros((N_VERTS, 5), f32)], axis=1)
    table = _pad_rows(table, NVP)

    gathered = _gather_sc(table, flat_idx2d)
    g3 = gathered[:NFLAT].reshape(3, NQP, 32)

    z8 = jnp.zeros((8, HID), f32)
    meW1p = jnp.concatenate([me_W1, z8], axis=0)
    peW1p = jnp.concatenate([pe_W1, jnp.zeros((1, HID), f32)], axis=0)
    mdW3p = jnp.concatenate([md_W3, jnp.zeros((HID, 1), f32)], axis=1)
    mdb3p = jnp.concatenate([md_b3, jnp.zeros((1,), f32)])[None]
    weights = [
        meW1p, me_b1[None], me_W2, me_b2[None],
        peW1p, pe_b1[None], pe_W2, pe_b2[None],
        md_W1[:HID], md_W1[HID:], md_b1[None], md_W2, md_b2[None],
        mdW3p, mdb3p,
        ld_W1[:HID], ld_W1[HID:], ld_b1[None], ld_W2, ld_b2[None],
        ld_W3, ld_b3[None],
    ]
    out48 = _dense(m8q, g3, weights)

    new_means = out48[:N_GAUSS, 0:3]
    new_quats = out48[:N_GAUSS, 3:7]
    new_features = out48[:N_GAUSS, 8:8 + LATENT]
    return (new_means, new_quats, new_features, jnp.float32(0.0))


# speedup vs baseline: 11.1243x; 1.1656x over previous
import functools

import jax
import jax.numpy as jnp
from jax import lax
from jax.experimental import pallas as pl
from jax.experimental.pallas import tpu as pltpu
from jax.experimental.pallas import tpu_sc as plsc

N_GAUSS = 50000
N_VERTS = 5143
WINDOW = 8
LATENT = 32
HID = 32

NQP = 50176
NVP = 5248
NFLAT = 3 * NQP
NW = 32
PER_W = 5120
NFLAT_PAD = NW * PER_W
CHUNK = 128
N_CHUNKS = PER_W // CHUNK

_BIG = 1e30


def _knn_body(m_ref, c_ref, idx_ref):
    d2 = jnp.dot(m_ref[...], c_ref[...], preferred_element_type=jnp.float32)
    rows = d2.shape[0]
    bigf = jnp.float32(_BIG)

    v1 = jnp.full((rows, 128), _BIG, jnp.float32)
    v2 = v1
    v3 = v1
    i1 = jnp.zeros((rows, 128), jnp.int32)
    i2 = i1
    i3 = i1
    for t in range(NVP // 128):
        x = lax.slice_in_dim(d2, t * 128, (t + 1) * 128, axis=1)
        tt = jnp.int32(t)
        c1 = x < v1
        nv1 = jnp.minimum(v1, x)
        da = jnp.maximum(v1, x)
        ni1 = jnp.where(c1, tt, i1)
        dia = jnp.where(c1, i1, tt)
        c2 = da < v2
        nv2 = jnp.minimum(v2, da)
        db = jnp.maximum(v2, da)
        ni2 = jnp.where(c2, dia, i2)
        dib = jnp.where(c2, i2, dia)
        c3 = db < v3
        nv3 = jnp.minimum(v3, db)
        ni3 = jnp.where(c3, dib, i3)
        v1, v2, v3, i1, i2, i3 = nv1, nv2, nv3, ni1, ni2, ni3

    lane = lax.broadcasted_iota(jnp.int32, (rows, 128), 1)
    g1 = i1 * 128 + lane
    g2 = i2 * 128 + lane
    g3 = i3 * 128 + lane
    picks = []
    for _ in range(3):
        mn = jnp.min(v1, axis=1, keepdims=True)
        p = jnp.min(jnp.where(v1 == mn, g1, jnp.int32(2**30)),
                    axis=1, keepdims=True)
        picks.append(p)
        hit = g1 == p
        v1 = jnp.where(hit, v2, v1)
        g1 = jnp.where(hit, g2, g1)
        v2 = jnp.where(hit, v3, v2)
        g2 = jnp.where(hit, g3, g2)
        v3 = jnp.where(hit, bigf, v3)

    lane8 = lax.broadcasted_iota(jnp.int32, (rows, 8), 1)
    out = jnp.where(lane8 == 0, picks[0],
                    jnp.where(lane8 == 1, picks[1],
                              jnp.where(lane8 == 2, picks[2], 0)))
    idx_ref[...] = out


def _knn(means8, cvt8):
    blk = 256
    return pl.pallas_call(
        _knn_body,
        grid=(NQP // blk,),
        in_specs=[
            pl.BlockSpec((blk, 8), lambda i: (i, 0)),
            pl.BlockSpec((8, NVP), lambda i: (0, 0)),
        ],
        out_specs=pl.BlockSpec((blk, 8), lambda i: (i, 0)),
        out_shape=jax.ShapeDtypeStruct((NQP, 8), jnp.int32),
    )(means8, cvt8)


def _gather_body(table_hbm, idx_hbm, out_hbm, idx_v, rows_a, rows_b, sem_a, sem_b):
    wid = lax.axis_index("s") * 2 + lax.axis_index("c")
    base = wid * PER_W
    pltpu.sync_copy(idx_hbm.at[pl.ds(wid * N_CHUNKS, N_CHUNKS)], idx_v)

    def pair(p, carry):
        c0 = 2 * p
        c1 = c0 + 1
        cp0 = pltpu.async_copy(table_hbm.at[idx_v.at[c0]], rows_a, sem_a)
        cp1 = pltpu.async_copy(table_hbm.at[idx_v.at[c1]], rows_b, sem_b)
        cp0.wait()
        pltpu.sync_copy(rows_a, out_hbm.at[pl.ds(base + c0 * CHUNK, CHUNK)])
        cp1.wait()
        pltpu.sync_copy(rows_b, out_hbm.at[pl.ds(base + c1 * CHUNK, CHUNK)])
        return carry

    lax.fori_loop(0, N_CHUNKS // 2, pair, 0)


def _gather_sc(table, flat_idx2d):
    mesh = plsc.VectorSubcoreMesh(core_axis_name="c", subcore_axis_name="s")
    run = functools.partial(
        pl.kernel,
        out_type=jax.ShapeDtypeStruct((NFLAT_PAD, 32), jnp.float32),
        mesh=mesh,
        scratch_types=[
            pltpu.VMEM((N_CHUNKS, CHUNK), jnp.int32),
            pltpu.VMEM((CHUNK, 32), jnp.float32),
            pltpu.VMEM((CHUNK, 32), jnp.float32),
            pltpu.SemaphoreType.DMA,
            pltpu.SemaphoreType.DMA,
        ],
        compiler_params=pltpu.CompilerParams(use_tc_tiling_on_sc=False),
    )(_gather_body)
    return run(table, flat_idx2d)


def _silu(x):
    return x / (1.0 + jnp.exp(-x))


def _dense_body(mq_ref, g_ref,
                meW1_ref, meb1_ref, meW2_ref, meb2_ref,
                peW1_ref, peb1_ref, peW2_ref, peb2_ref,
                mdW1a_ref, mdW1b_ref, mdb1_ref, mdW2_ref, mdb2_ref,
                mdW3_ref, mdb3_ref,
                ldW1a_ref, ldW1b_ref, ldb1_ref, ldW2_ref, ldb2_ref,
                ldW3_ref, ldb3_ref,
                out_ref):
    mq = mq_ref[...]
    g0 = g_ref[0]
    g1 = g_ref[1]
    g2 = g_ref[2]
    rows = mq.shape[0]

    means3 = mq[:, 0:3]
    v0 = g0[:, 24:27]
    e1 = g1[:, 24:27] - v0
    e2 = g2[:, 24:27] - v0
    ep = means3 - v0
    d00 = jnp.sum(e1 * e1, axis=1, keepdims=True)
    d01 = jnp.sum(e1 * e2, axis=1, keepdims=True)
    d11 = jnp.sum(e2 * e2, axis=1, keepdims=True)
    d20 = jnp.sum(ep * e1, axis=1, keepdims=True)
    d21 = jnp.sum(ep * e2, axis=1, keepdims=True)
    denom = d00 * d11 - d01 * d01 + 1e-8
    v = (d11 * d20 - d01 * d21) / denom
    w = (d00 * d21 - d01 * d20) / denom
    u = 1.0 - v - w

    nm = u * g0 + v * g1 + w * g2

    def mm(a, w_ref):
        return jnp.dot(a, w_ref[...], preferred_element_type=jnp.float32)

    x = _silu(mm(_silu(mm(nm, meW1_ref) + meb1_ref[...]), meW2_ref) + meb2_ref[...])
    pe = _silu(mm(_silu(mm(mq, peW1_ref) + peb1_ref[...]), peW2_ref) + peb2_ref[...])

    h = _silu(mm(x, mdW1a_ref) + mm(pe, mdW1b_ref) + mdb1_ref[...])
    h = _silu(mm(h, mdW2_ref) + mdb2_ref[...])
    mv = mm(h, mdW3_ref) + mdb3_ref[...]

    lane = lax.broadcasted_iota(jnp.int32, (rows, 8), 1)
    scale = jnp.where(lane < 3, jnp.float32(0.001),
                      jnp.where(lane < 7, jnp.float32(0.01), jnp.float32(0.0)))
    upd8 = mq + scale * mv

    l = _silu(mm(x, ldW1a_ref) + mm(pe, ldW1b_ref) + ldb1_ref[...])
    l = _silu(mm(l, ldW2_ref) + ldb2_ref[...])
    feat = mm(l, ldW3_ref) + ldb3_ref[...]

    out_ref[...] = jnp.concatenate(
        [upd8, feat, jnp.zeros((rows, 8), jnp.float32)], axis=1)


def _dense(mq, g3, weights):
    blk = 256
    w_specs = [pl.BlockSpec(w.shape, lambda i: tuple(0 for _ in w.shape))
               for w in weights]
    return pl.pallas_call(
        _dense_body,
        grid=(NQP // blk,),
        in_specs=[
            pl.BlockSpec((blk, 8), lambda i: (i, 0)),
            pl.BlockSpec((3, blk, 32), lambda i: (0, i, 0)),
        ] + w_specs,
        out_specs=pl.BlockSpec((blk, 48), lambda i: (i, 0)),
        out_shape=jax.ShapeDtypeStruct((NQP, 48), jnp.float32),
    )(mq, g3, *weights)


def _pad_rows(a, n):
    return jnp.concatenate(
        [a, jnp.zeros((n - a.shape[0],) + a.shape[1:], a.dtype)], axis=0)


def kernel(means, quats, features, flame_vertices, canonical_vertices,
           me_W1, me_b1, me_W2, me_b2, pe_W1, pe_b1, pe_W2, pe_b2,
           md_W1, md_b1, md_W2, md_b2, md_W3, md_b3,
           ld_W1, ld_b1, ld_W2, ld_b2, ld_W3, ld_b3):
    f32 = jnp.float32

    m8q = jnp.concatenate(
        [_pad_rows(means, NQP), _pad_rows(quats, NQP), jnp.ones((NQP, 1), f32)],
        axis=1)
    cn2 = jnp.sum(canonical_vertices * canonical_vertices, axis=1)
    top = jnp.concatenate(
        [-2.0 * canonical_vertices.T, jnp.zeros((4, N_VERTS), f32), cn2[None]],
        axis=0)
    padcols = jnp.concatenate(
        [jnp.zeros((7, NVP - N_VERTS), f32),
         jnp.full((1, NVP - N_VERTS), _BIG, f32)], axis=0)
    cvt8 = jnp.concatenate([top, padcols], axis=1)

    idx8 = _knn(m8q, cvt8)

    flat_idx = idx8[:, :3].T.reshape(NFLAT)
    flat_idx = jnp.concatenate(
        [flat_idx, jnp.zeros((NFLAT_PAD - NFLAT,), jnp.int32)])
    flat_idx2d = flat_idx.reshape(NFLAT_PAD // CHUNK, CHUNK)

    vm = jnp.transpose(flame_vertices, (1, 0, 2)).reshape(N_VERTS, WINDOW * 3)
    table = jnp.concatenate(
        [vm, canonical_vertices, jnp.zeros((N_VERTS, 5), f32)], axis=1)
    table = _pad_rows(table, NVP)

    gathered = _gather_sc(table, flat_idx2d)
    g3 = gathered[:NFLAT].reshape(3, NQP, 32)

    z8 = jnp.zeros((8, HID), f32)
    meW1p = jnp.concatenate([me_W1, z8], axis=0)
    peW1p = jnp.concatenate([pe_W1, jnp.zeros((1, HID), f32)], axis=0)
    mdW3p = jnp.concatenate([md_W3, jnp.zeros((HID, 1), f32)], axis=1)
    mdb3p = jnp.concatenate([md_b3, jnp.zeros((1,), f32)])[None]
    weights = [
        meW1p, me_b1[None], me_W2, me_b2[None],
        peW1p, pe_b1[None], pe_W2, pe_b2[None],
        md_W1[:HID], md_W1[HID:], md_b1[None], md_W2, md_b2[None],
        mdW3p, mdb3p,
        ld_W1[:HID], ld_W1[HID:], ld_b1[None], ld_W2, ld_b2[None],
        ld_W3, ld_b3[None],
    ]
    out48 = _dense(m8q, g3, weights)

    new_means = out48[:N_GAUSS, 0:3]
    new_quats = out48[:N_GAUSS, 3:7]
    new_features = out48[:N_GAUSS, 8:8 + LATENT]
    return (new_means, new_quats, new_features, jnp.float32(0.0))


# speedup vs baseline: 14.1440x; 1.2715x over previous
import functools

import jax
import jax.numpy as jnp
from jax import lax
from jax.experimental import pallas as pl
from jax.experimental.pallas import tpu as pltpu
from jax.experimental.pallas import tpu_sc as plsc

N_GAUSS = 50000
N_VERTS = 5143
WINDOW = 8
LATENT = 32
HID = 32

NQP = 50176
NVP = 5248
NW = 32
Q_PER_W = NQP // NW
CHUNK = 128
N_FULL = Q_PER_W // CHUNK
TAIL = Q_PER_W - N_FULL * CHUNK

_BIG = 1e30


def _top3_subblock(d2):
    rows = d2.shape[0]
    bigf = jnp.float32(_BIG)
    v1 = jnp.full((rows, 128), _BIG, jnp.float32)
    v2 = v1
    v3 = v1
    i1 = jnp.zeros((rows, 128), jnp.int32)
    i2 = i1
    i3 = i1
    for t in range(NVP // 128):
        x = lax.slice_in_dim(d2, t * 128, (t + 1) * 128, axis=1)
        tt = jnp.int32(t)
        c1 = x < v1
        nv1 = jnp.minimum(v1, x)
        da = jnp.maximum(v1, x)
        ni1 = jnp.where(c1, tt, i1)
        dia = jnp.where(c1, i1, tt)
        c2 = da < v2
        nv2 = jnp.minimum(v2, da)
        db = jnp.maximum(v2, da)
        ni2 = jnp.where(c2, dia, i2)
        dib = jnp.where(c2, i2, dia)
        c3 = db < v3
        nv3 = jnp.minimum(v3, db)
        ni3 = jnp.where(c3, dib, i3)
        v1, v2, v3, i1, i2, i3 = nv1, nv2, nv3, ni1, ni2, ni3

    lane = lax.broadcasted_iota(jnp.int32, (rows, 128), 1)
    g1 = i1 * 128 + lane
    g2 = i2 * 128 + lane
    g3 = i3 * 128 + lane
    picks = []
    for _ in range(3):
        mn = jnp.min(v1, axis=1, keepdims=True)
        p = jnp.min(jnp.where(v1 == mn, g1, jnp.int32(2**30)),
                    axis=1, keepdims=True)
        picks.append(p)
        hit = g1 == p
        v1 = jnp.where(hit, v2, v1)
        g1 = jnp.where(hit, g2, g1)
        v2 = jnp.where(hit, v3, v2)
        g2 = jnp.where(hit, g3, g2)
        v3 = jnp.where(hit, bigf, v3)

    lane8 = lax.broadcasted_iota(jnp.int32, (rows, 8), 1)
    return jnp.where(lane8 == 0, picks[0],
                     jnp.where(lane8 == 1, picks[1],
                               jnp.where(lane8 == 2, picks[2], 0)))


def _knn_body(m_ref, c_ref, idx_ref):
    d2 = jnp.dot(m_ref[...], c_ref[...], preferred_element_type=jnp.float32)
    idx_ref[...] = _top3_subblock(d2)


def _knn(means8, cvt8):
    blk = 512
    return pl.pallas_call(
        _knn_body,
        grid=(NQP // blk,),
        in_specs=[
            pl.BlockSpec((blk, 8), lambda i: (i, 0)),
            pl.BlockSpec((8, NVP), lambda i: (0, 0)),
        ],
        out_specs=pl.BlockSpec((blk, 8), lambda i: (i, 0)),
        out_shape=jax.ShapeDtypeStruct((NQP, 8), jnp.int32),
    )(means8, cvt8)


def _gather_body(table_hbm, idx8_hbm, out_hbm,
                 idxblk, list0, list1, list2, rows_a, rows_b, sem_a, sem_b):
    wid = lax.axis_index("s") * 2 + lax.axis_index("c")
    qbase = wid * Q_PER_W
    pltpu.sync_copy(idx8_hbm.at[pl.ds(qbase, Q_PER_W)], idxblk)

    iota16 = lax.broadcasted_iota(jnp.int32, (16,), 0)
    for k, listk in ((0, list0), (1, list1), (2, list2)):
        ksplat = jnp.full((16,), k, jnp.int32)

        def build(j, carry, listk=listk, ksplat=ksplat):
            rows16 = j * 16 + iota16
            listk[pl.ds(j * 16, 16)] = plsc.load_gather(idxblk, [rows16, ksplat])
            return carry

        lax.fori_loop(0, Q_PER_W // 16, build, 0)

    for k, listk in ((0, list0), (1, list1), (2, list2)):

        def pair(p, carry, k=k, listk=listk):
            c0 = 2 * p * CHUNK
            c1 = c0 + CHUNK
            cp0 = pltpu.async_copy(table_hbm.at[listk.at[pl.ds(c0, CHUNK)]],
                                   rows_a, sem_a)
            cp1 = pltpu.async_copy(table_hbm.at[listk.at[pl.ds(c1, CHUNK)]],
                                   rows_b, sem_b)
            cp0.wait()
            pltpu.sync_copy(rows_a, out_hbm.at[k, pl.ds(qbase + c0, CHUNK)])
            cp1.wait()
            pltpu.sync_copy(rows_b, out_hbm.at[k, pl.ds(qbase + c1, CHUNK)])
            return carry

        lax.fori_loop(0, N_FULL // 2, pair, 0)
        tbase = N_FULL * CHUNK
        cpt = pltpu.async_copy(table_hbm.at[listk.at[pl.ds(tbase, TAIL)]],
                               rows_a.at[pl.ds(0, TAIL)], sem_a)
        cpt.wait()
        pltpu.sync_copy(rows_a.at[pl.ds(0, TAIL)],
                        out_hbm.at[k, pl.ds(qbase + tbase, TAIL)])


def _gather_sc(table, idx8):
    mesh = plsc.VectorSubcoreMesh(core_axis_name="c", subcore_axis_name="s")
    run = functools.partial(
        pl.kernel,
        out_type=jax.ShapeDtypeStruct((3, NQP, 32), jnp.float32),
        mesh=mesh,
        scratch_types=[
            pltpu.VMEM((Q_PER_W, 8), jnp.int32),
            pltpu.VMEM((Q_PER_W,), jnp.int32),
            pltpu.VMEM((Q_PER_W,), jnp.int32),
            pltpu.VMEM((Q_PER_W,), jnp.int32),
            pltpu.VMEM((CHUNK, 32), jnp.float32),
            pltpu.VMEM((CHUNK, 32), jnp.float32),
            pltpu.SemaphoreType.DMA,
            pltpu.SemaphoreType.DMA,
        ],
        compiler_params=pltpu.CompilerParams(use_tc_tiling_on_sc=False,
                                             needs_layout_passes=False),
    )(_gather_body)
    return run(table, idx8)


def _silu(x):
    return x / (1.0 + jnp.exp(-x))


def _dense_body(mq_ref, g_ref,
                meW1_ref, meb1_ref, meW2_ref, meb2_ref,
                peW1_ref, peb1_ref, peW2_ref, peb2_ref,
                mdW1a_ref, mdW1b_ref, mdb1_ref, mdW2_ref, mdb2_ref,
                mdW3_ref, mdb3_ref,
                ldW1a_ref, ldW1b_ref, ldb1_ref, ldW2_ref, ldb2_ref,
                ldW3_ref, ldb3_ref,
                out_ref):
    mq = mq_ref[...]
    g0 = g_ref[0]
    g1 = g_ref[1]
    g2 = g_ref[2]
    rows = mq.shape[0]

    means3 = mq[:, 0:3]
    v0 = g0[:, 24:27]
    e1 = g1[:, 24:27] - v0
    e2 = g2[:, 24:27] - v0
    ep = means3 - v0
    d00 = jnp.sum(e1 * e1, axis=1, keepdims=True)
    d01 = jnp.sum(e1 * e2, axis=1, keepdims=True)
    d11 = jnp.sum(e2 * e2, axis=1, keepdims=True)
    d20 = jnp.sum(ep * e1, axis=1, keepdims=True)
    d21 = jnp.sum(ep * e2, axis=1, keepdims=True)
    denom = d00 * d11 - d01 * d01 + 1e-8
    v = (d11 * d20 - d01 * d21) / denom
    w = (d00 * d21 - d01 * d20) / denom
    u = 1.0 - v - w

    nm = u * g0 + v * g1 + w * g2

    def mm(a, w_ref):
        return jnp.dot(a, w_ref[...], preferred_element_type=jnp.float32)

    x = _silu(mm(_silu(mm(nm, meW1_ref) + meb1_ref[...]), meW2_ref) + meb2_ref[...])
    pe = _silu(mm(_silu(mm(mq, peW1_ref) + peb1_ref[...]), peW2_ref) + peb2_ref[...])

    h = _silu(mm(x, mdW1a_ref) + mm(pe, mdW1b_ref) + mdb1_ref[...])
    h = _silu(mm(h, mdW2_ref) + mdb2_ref[...])
    mv = mm(h, mdW3_ref) + mdb3_ref[...]

    lane = lax.broadcasted_iota(jnp.int32, (rows, 8), 1)
    scale = jnp.where(lane < 3, jnp.float32(0.001),
                      jnp.where(lane < 7, jnp.float32(0.01), jnp.float32(0.0)))
    upd8 = mq + scale * mv

    l = _silu(mm(x, ldW1a_ref) + mm(pe, ldW1b_ref) + ldb1_ref[...])
    l = _silu(mm(l, ldW2_ref) + ldb2_ref[...])
    feat = mm(l, ldW3_ref) + ldb3_ref[...]

    out_ref[...] = jnp.concatenate(
        [upd8, feat, jnp.zeros((rows, 8), jnp.float32)], axis=1)


def _dense(mq, g3, weights):
    blk = 512
    w_specs = [pl.BlockSpec(w.shape, lambda i: tuple(0 for _ in w.shape))
               for w in weights]
    return pl.pallas_call(
        _dense_body,
        grid=(NQP // blk,),
        in_specs=[
            pl.BlockSpec((blk, 8), lambda i: (i, 0)),
            pl.BlockSpec((3, blk, 32), lambda i: (0, i, 0)),
        ] + w_specs,
        out_specs=pl.BlockSpec((blk, 48), lambda i: (i, 0)),
        out_shape=jax.ShapeDtypeStruct((NQP, 48), jnp.float32),
    )(mq, g3, *weights)


def _pad_rows(a, n):
    return jnp.concatenate(
        [a, jnp.zeros((n - a.shape[0],) + a.shape[1:], a.dtype)], axis=0)


def kernel(means, quats, features, flame_vertices, canonical_vertices,
           me_W1, me_b1, me_W2, me_b2, pe_W1, pe_b1, pe_W2, pe_b2,
           md_W1, md_b1, md_W2, md_b2, md_W3, md_b3,
           ld_W1, ld_b1, ld_W2, ld_b2, ld_W3, ld_b3):
    f32 = jnp.float32

    m8q = jnp.concatenate(
        [_pad_rows(means, NQP), _pad_rows(quats, NQP), jnp.ones((NQP, 1), f32)],
        axis=1)
    cn2 = jnp.sum(canonical_vertices * canonical_vertices, axis=1)
    top = jnp.concatenate(
        [-2.0 * canonical_vertices.T, jnp.zeros((4, N_VERTS), f32), cn2[None]],
        axis=0)
    padcols = jnp.concatenate(
        [jnp.zeros((7, NVP - N_VERTS), f32),
         jnp.full((1, NVP - N_VERTS), _BIG, f32)], axis=0)
    cvt8 = jnp.concatenate([top, padcols], axis=1)

    idx8 = _knn(m8q, cvt8)

    vm = jnp.transpose(flame_vertices, (1, 0, 2)).reshape(N_VERTS, WINDOW * 3)
    table = jnp.concatenate(
        [vm, canonical_vertices, jnp.zeros((N_VERTS, 5), f32)], axis=1)
    table = _pad_rows(table, NVP)

    g3 = _gather_sc(table, idx8)

    z8 = jnp.zeros((8, HID), f32)
    meW1p = jnp.concatenate([me_W1, z8], axis=0)
    peW1p = jnp.concatenate([pe_W1, jnp.zeros((1, HID), f32)], axis=0)
    mdW3p = jnp.concatenate([md_W3, jnp.zeros((HID, 1), f32)], axis=1)
    mdb3p = jnp.concatenate([md_b3, jnp.zeros((1,), f32)])[None]
    weights = [
        meW1p, me_b1[None], me_W2, me_b2[None],
        peW1p, pe_b1[None], pe_W2, pe_b2[None],
        md_W1[:HID], md_W1[HID:], md_b1[None], md_W2, md_b2[None],
        mdW3p, mdb3p,
        ld_W1[:HID], ld_W1[HID:], ld_b1[None], ld_W2, ld_b2[None],
        ld_W3, ld_b3[None],
    ]
    out48 = _dense(m8q, g3, weights)

    new_means = out48[:N_GAUSS, 0:3]
    new_quats = out48[:N_GAUSS, 3:7]
    new_features = out48[:N_GAUSS, 8:8 + LATENT]
    return (new_means, new_quats, new_features, jnp.float32(0.0))


# speedup vs baseline: 14.8187x; 1.0477x over previous
import functools

import jax
import jax.numpy as jnp
from jax import lax
from jax.experimental import pallas as pl
from jax.experimental.pallas import tpu as pltpu
from jax.experimental.pallas import tpu_sc as plsc

N_GAUSS = 50000
N_VERTS = 5143
WINDOW = 8
LATENT = 32
HID = 32

NQP = 50176
NVP = 5248
NW = 32
Q_PER_W = NQP // NW
CHUNK = 128
N_FULL = Q_PER_W // CHUNK
TAIL = Q_PER_W - N_FULL * CHUNK

_BIG = 1e30


def _top3_subblock(d2):
    rows = d2.shape[0]
    bigf = jnp.float32(_BIG)
    v1 = jnp.full((rows, 128), _BIG, jnp.float32)
    v2 = v1
    v3 = v1
    i1 = jnp.zeros((rows, 128), jnp.int32)
    i2 = i1
    i3 = i1
    for t in range(NVP // 128):
        x = lax.slice_in_dim(d2, t * 128, (t + 1) * 128, axis=1)
        tt = jnp.int32(t)
        c1 = x < v1
        nv1 = jnp.minimum(v1, x)
        da = jnp.maximum(v1, x)
        ni1 = jnp.where(c1, tt, i1)
        dia = jnp.where(c1, i1, tt)
        c2 = da < v2
        nv2 = jnp.minimum(v2, da)
        db = jnp.maximum(v2, da)
        ni2 = jnp.where(c2, dia, i2)
        dib = jnp.where(c2, i2, dia)
        c3 = db < v3
        nv3 = jnp.minimum(v3, db)
        ni3 = jnp.where(c3, dib, i3)
        v1, v2, v3, i1, i2, i3 = nv1, nv2, nv3, ni1, ni2, ni3

    lane = lax.broadcasted_iota(jnp.int32, (rows, 128), 1)
    g1 = i1 * 128 + lane
    g2 = i2 * 128 + lane
    g3 = i3 * 128 + lane
    picks = []
    for _ in range(3):
        mn = jnp.min(v1, axis=1, keepdims=True)
        p = jnp.min(jnp.where(v1 == mn, g1, jnp.int32(2**30)),
                    axis=1, keepdims=True)
        picks.append(p)
        hit = g1 == p
        v1 = jnp.where(hit, v2, v1)
        g1 = jnp.where(hit, g2, g1)
        v2 = jnp.where(hit, v3, v2)
        g2 = jnp.where(hit, g3, g2)
        v3 = jnp.where(hit, bigf, v3)

    lane8 = lax.broadcasted_iota(jnp.int32, (rows, 8), 1)
    return jnp.where(lane8 == 0, picks[0],
                     jnp.where(lane8 == 1, picks[1],
                               jnp.where(lane8 == 2, picks[2], 0)))


def _knn_body(m_ref, c_ref, idx_ref):
    d2 = jnp.dot(m_ref[...], c_ref[...], preferred_element_type=jnp.float32)
    idx_ref[...] = _top3_subblock(d2)


def _knn(means8, cvt8):
    blk = 1024
    return pl.pallas_call(
        _knn_body,
        grid=(NQP // blk,),
        in_specs=[
            pl.BlockSpec((blk, 8), lambda i: (i, 0)),
            pl.BlockSpec((8, NVP), lambda i: (0, 0)),
        ],
        out_specs=pl.BlockSpec((blk, 8), lambda i: (i, 0)),
        out_shape=jax.ShapeDtypeStruct((NQP, 8), jnp.int32),
    )(means8, cvt8)


def _gather_body(table_hbm, idx8_hbm, out_hbm,
                 idxblk, list0, list1, list2, rows_a, rows_b, sem_a, sem_b):
    wid = lax.axis_index("s") * 2 + lax.axis_index("c")
    qbase = wid * Q_PER_W
    pltpu.sync_copy(idx8_hbm.at[pl.ds(qbase, Q_PER_W)], idxblk)

    iota16 = lax.broadcasted_iota(jnp.int32, (16,), 0)
    for k, listk in ((0, list0), (1, list1), (2, list2)):
        ksplat = jnp.full((16,), k, jnp.int32)

        def build(j, carry, listk=listk, ksplat=ksplat):
            rows16 = j * 16 + iota16
            listk[pl.ds(j * 16, 16)] = plsc.load_gather(idxblk, [rows16, ksplat])
            return carry

        lax.fori_loop(0, Q_PER_W // 16, build, 0)

    for k, listk in ((0, list0), (1, list1), (2, list2)):

        def pair(p, carry, k=k, listk=listk):
            c0 = 2 * p * CHUNK
            c1 = c0 + CHUNK
            cp0 = pltpu.async_copy(table_hbm.at[listk.at[pl.ds(c0, CHUNK)]],
                                   rows_a, sem_a)
            cp1 = pltpu.async_copy(table_hbm.at[listk.at[pl.ds(c1, CHUNK)]],
                                   rows_b, sem_b)
            cp0.wait()
            pltpu.sync_copy(rows_a, out_hbm.at[k, pl.ds(qbase + c0, CHUNK)])
            cp1.wait()
            pltpu.sync_copy(rows_b, out_hbm.at[k, pl.ds(qbase + c1, CHUNK)])
            return carry

        lax.fori_loop(0, N_FULL // 2, pair, 0)
        tbase = N_FULL * CHUNK
        cpt = pltpu.async_copy(table_hbm.at[listk.at[pl.ds(tbase, TAIL)]],
                               rows_a.at[pl.ds(0, TAIL)], sem_a)
        cpt.wait()
        pltpu.sync_copy(rows_a.at[pl.ds(0, TAIL)],
                        out_hbm.at[k, pl.ds(qbase + tbase, TAIL)])


def _gather_sc(table, idx8):
    mesh = plsc.VectorSubcoreMesh(core_axis_name="c", subcore_axis_name="s")
    run = functools.partial(
        pl.kernel,
        out_type=jax.ShapeDtypeStruct((3, NQP, 32), jnp.float32),
        mesh=mesh,
        scratch_types=[
            pltpu.VMEM((Q_PER_W, 8), jnp.int32),
            pltpu.VMEM((Q_PER_W,), jnp.int32),
            pltpu.VMEM((Q_PER_W,), jnp.int32),
            pltpu.VMEM((Q_PER_W,), jnp.int32),
            pltpu.VMEM((CHUNK, 32), jnp.float32),
            pltpu.VMEM((CHUNK, 32), jnp.float32),
            pltpu.SemaphoreType.DMA,
            pltpu.SemaphoreType.DMA,
        ],
        compiler_params=pltpu.CompilerParams(use_tc_tiling_on_sc=False,
                                             needs_layout_passes=False),
    )(_gather_body)
    return run(table, idx8)


def _silu(x):
    return x / (1.0 + jnp.exp(-x))


def _dense_body(mq_ref, g_ref,
                meW1_ref, meb1_ref, meW2_ref, meb2_ref,
                peW1_ref, peb1_ref, peW2_ref, peb2_ref,
                mdW1a_ref, mdW1b_ref, mdb1_ref, mdW2_ref, mdb2_ref,
                mdW3_ref, mdb3_ref,
                ldW1a_ref, ldW1b_ref, ldb1_ref, ldW2_ref, ldb2_ref,
                ldW3_ref, ldb3_ref,
                out_ref):
    mq = mq_ref[...]
    g0 = g_ref[0]
    g1 = g_ref[1]
    g2 = g_ref[2]
    rows = mq.shape[0]

    means3 = mq[:, 0:3]
    v0 = g0[:, 24:27]
    e1 = g1[:, 24:27] - v0
    e2 = g2[:, 24:27] - v0
    ep = means3 - v0
    d00 = jnp.sum(e1 * e1, axis=1, keepdims=True)
    d01 = jnp.sum(e1 * e2, axis=1, keepdims=True)
    d11 = jnp.sum(e2 * e2, axis=1, keepdims=True)
    d20 = jnp.sum(ep * e1, axis=1, keepdims=True)
    d21 = jnp.sum(ep * e2, axis=1, keepdims=True)
    denom = d00 * d11 - d01 * d01 + 1e-8
    v = (d11 * d20 - d01 * d21) / denom
    w = (d00 * d21 - d01 * d20) / denom
    u = 1.0 - v - w

    nm = u * g0 + v * g1 + w * g2

    def mm(a, w_ref):
        return jnp.dot(a, w_ref[...], preferred_element_type=jnp.float32)

    x = _silu(mm(_silu(mm(nm, meW1_ref) + meb1_ref[...]), meW2_ref) + meb2_ref[...])
    pe = _silu(mm(_silu(mm(mq, peW1_ref) + peb1_ref[...]), peW2_ref) + peb2_ref[...])

    h = _silu(mm(x, mdW1a_ref) + mm(pe, mdW1b_ref) + mdb1_ref[...])
    h = _silu(mm(h, mdW2_ref) + mdb2_ref[...])
    mv = mm(h, mdW3_ref) + mdb3_ref[...]

    lane = lax.broadcasted_iota(jnp.int32, (rows, 8), 1)
    scale = jnp.where(lane < 3, jnp.float32(0.001),
                      jnp.where(lane < 7, jnp.float32(0.01), jnp.float32(0.0)))
    upd8 = mq + scale * mv

    l = _silu(mm(x, ldW1a_ref) + mm(pe, ldW1b_ref) + ldb1_ref[...])
    l = _silu(mm(l, ldW2_ref) + ldb2_ref[...])
    feat = mm(l, ldW3_ref) + ldb3_ref[...]

    out_ref[...] = jnp.concatenate(
        [upd8, feat, jnp.zeros((rows, 8), jnp.float32)], axis=1)


def _dense(mq, g3, weights):
    blk = 1024
    w_specs = [pl.BlockSpec(w.shape, lambda i: tuple(0 for _ in w.shape))
               for w in weights]
    return pl.pallas_call(
        _dense_body,
        grid=(NQP // blk,),
        in_specs=[
            pl.BlockSpec((blk, 8), lambda i: (i, 0)),
            pl.BlockSpec((3, blk, 32), lambda i: (0, i, 0)),
        ] + w_specs,
        out_specs=pl.BlockSpec((blk, 48), lambda i: (i, 0)),
        out_shape=jax.ShapeDtypeStruct((NQP, 48), jnp.float32),
    )(mq, g3, *weights)


def _pad_rows(a, n):
    return jnp.concatenate(
        [a, jnp.zeros((n - a.shape[0],) + a.shape[1:], a.dtype)], axis=0)


def kernel(means, quats, features, flame_vertices, canonical_vertices,
           me_W1, me_b1, me_W2, me_b2, pe_W1, pe_b1, pe_W2, pe_b2,
           md_W1, md_b1, md_W2, md_b2, md_W3, md_b3,
           ld_W1, ld_b1, ld_W2, ld_b2, ld_W3, ld_b3):
    f32 = jnp.float32

    m8q = jnp.concatenate(
        [_pad_rows(means, NQP), _pad_rows(quats, NQP), jnp.ones((NQP, 1), f32)],
        axis=1)
    cn2 = jnp.sum(canonical_vertices * canonical_vertices, axis=1)
    top = jnp.concatenate(
        [-2.0 * canonical_vertices.T, jnp.zeros((4, N_VERTS), f32), cn2[None]],
        axis=0)
    padcols = jnp.concatenate(
        [jnp.zeros((7, NVP - N_VERTS), f32),
         jnp.full((1, NVP - N_VERTS), _BIG, f32)], axis=0)
    cvt8 = jnp.concatenate([top, padcols], axis=1)

    idx8 = _knn(m8q, cvt8)

    vm = jnp.transpose(flame_vertices, (1, 0, 2)).reshape(N_VERTS, WINDOW * 3)
    table = jnp.concatenate(
        [vm, canonical_vertices, jnp.zeros((N_VERTS, 5), f32)], axis=1)
    table = _pad_rows(table, NVP)

    g3 = _gather_sc(table, idx8)

    z8 = jnp.zeros((8, HID), f32)
    meW1p = jnp.concatenate([me_W1, z8], axis=0)
    peW1p = jnp.concatenate([pe_W1, jnp.zeros((1, HID), f32)], axis=0)
    mdW3p = jnp.concatenate([md_W3, jnp.zeros((HID, 1), f32)], axis=1)
    mdb3p = jnp.concatenate([md_b3, jnp.zeros((1,), f32)])[None]
    weights = [
        meW1p, me_b1[None], me_W2, me_b2[None],
        peW1p, pe_b1[None], pe_W2, pe_b2[None],
        md_W1[:HID], md_W1[HID:], md_b1[None], md_W2, md_b2[None],
        mdW3p, mdb3p,
        ld_W1[:HID], ld_W1[HID:], ld_b1[None], ld_W2, ld_b2[None],
        ld_W3, ld_b3[None],
    ]
    out48 = _dense(m8q, g3, weights)

    new_means = out48[:N_GAUSS, 0:3]
    new_quats = out48[:N_GAUSS, 3:7]
    new_features = out48[:N_GAUSS, 8:8 + LATENT]
    return (new_means, new_quats, new_features, jnp.float32(0.0))


# speedup vs baseline: 15.4229x; 1.0408x over previous
import functools

import jax
import jax.numpy as jnp
from jax import lax
from jax.experimental import pallas as pl
from jax.experimental.pallas import tpu as pltpu
from jax.experimental.pallas import tpu_sc as plsc

N_GAUSS = 50000
N_VERTS = 5143
WINDOW = 8
LATENT = 32
HID = 32

NQP = 50176
NVP = 5248
NW = 32
Q_PER_W = NQP // NW
CHUNK = 128
N_FULL = Q_PER_W // CHUNK
TAIL = Q_PER_W - N_FULL * CHUNK

_BIG = 1e30


def _top3_subblock(d2):
    rows = d2.shape[0]
    bigf = jnp.float32(_BIG)
    v1 = jnp.full((rows, 128), _BIG, jnp.float32)
    v2 = v1
    v3 = v1
    i1 = jnp.zeros((rows, 128), jnp.int32)
    i2 = i1
    i3 = i1
    for t in range(NVP // 128):
        x = lax.slice_in_dim(d2, t * 128, (t + 1) * 128, axis=1)
        tt = jnp.int32(t)
        c1 = x < v1
        nv1 = jnp.minimum(v1, x)
        da = jnp.maximum(v1, x)
        ni1 = jnp.where(c1, tt, i1)
        dia = jnp.where(c1, i1, tt)
        c2 = da < v2
        nv2 = jnp.minimum(v2, da)
        db = jnp.maximum(v2, da)
        ni2 = jnp.where(c2, dia, i2)
        dib = jnp.where(c2, i2, dia)
        c3 = db < v3
        nv3 = jnp.minimum(v3, db)
        ni3 = jnp.where(c3, dib, i3)
        v1, v2, v3, i1, i2, i3 = nv1, nv2, nv3, ni1, ni2, ni3

    lane = lax.broadcasted_iota(jnp.int32, (rows, 128), 1)
    g1 = i1 * 128 + lane
    g2 = i2 * 128 + lane
    g3 = i3 * 128 + lane
    picks = []
    for _ in range(3):
        mn = jnp.min(v1, axis=1, keepdims=True)
        p = jnp.min(jnp.where(v1 == mn, g1, jnp.int32(2**30)),
                    axis=1, keepdims=True)
        picks.append(p)
        hit = g1 == p
        v1 = jnp.where(hit, v2, v1)
        g1 = jnp.where(hit, g2, g1)
        v2 = jnp.where(hit, v3, v2)
        g2 = jnp.where(hit, g3, g2)
        v3 = jnp.where(hit, bigf, v3)

    lane8 = lax.broadcasted_iota(jnp.int32, (rows, 8), 1)
    return jnp.where(lane8 == 0, picks[0],
                     jnp.where(lane8 == 1, picks[1],
                               jnp.where(lane8 == 2, picks[2], 0)))


def _knn_body(m_ref, c_ref, idx_ref):
    rows = m_ref.shape[0]
    m8 = jnp.concatenate(
        [m_ref[...], jnp.zeros((rows, 4), jnp.float32),
         jnp.ones((rows, 1), jnp.float32)], axis=1)
    d2 = jnp.dot(m8, c_ref[...], preferred_element_type=jnp.float32)
    idx_ref[...] = _top3_subblock(d2)


def _knn(means, cvt8):
    blk = 1024
    return pl.pallas_call(
        _knn_body,
        grid=(NQP // blk,),
        in_specs=[
            pl.BlockSpec((blk, 3), lambda i: (i, 0)),
            pl.BlockSpec((8, NVP), lambda i: (0, 0)),
        ],
        out_specs=pl.BlockSpec((blk, 8), lambda i: (i, 0)),
        out_shape=jax.ShapeDtypeStruct((NQP, 8), jnp.int32),
    )(means, cvt8)


def _gather_body(table_hbm, idx8_hbm, out_hbm,
                 idxblk, list0, list1, list2, rows_a, rows_b, sem_a, sem_b):
    wid = lax.axis_index("s") * 2 + lax.axis_index("c")
    qbase = wid * Q_PER_W
    pltpu.sync_copy(idx8_hbm.at[pl.ds(qbase, Q_PER_W)], idxblk)

    iota16 = lax.broadcasted_iota(jnp.int32, (16,), 0)
    for k, listk in ((0, list0), (1, list1), (2, list2)):
        ksplat = jnp.full((16,), k, jnp.int32)

        def build(j, carry, listk=listk, ksplat=ksplat):
            rows16 = j * 16 + iota16
            listk[pl.ds(j * 16, 16)] = plsc.load_gather(idxblk, [rows16, ksplat])
            return carry

        lax.fori_loop(0, Q_PER_W // 16, build, 0)

    for k, listk in ((0, list0), (1, list1), (2, list2)):

        def pair(p, carry, k=k, listk=listk):
            c0 = 2 * p * CHUNK
            c1 = c0 + CHUNK
            cp0 = pltpu.async_copy(table_hbm.at[listk.at[pl.ds(c0, CHUNK)]],
                                   rows_a, sem_a)
            cp1 = pltpu.async_copy(table_hbm.at[listk.at[pl.ds(c1, CHUNK)]],
                                   rows_b, sem_b)
            cp0.wait()
            pltpu.sync_copy(rows_a, out_hbm.at[k, pl.ds(qbase + c0, CHUNK)])
            cp1.wait()
            pltpu.sync_copy(rows_b, out_hbm.at[k, pl.ds(qbase + c1, CHUNK)])
            return carry

        lax.fori_loop(0, N_FULL // 2, pair, 0)
        tbase = N_FULL * CHUNK
        cpt = pltpu.async_copy(table_hbm.at[listk.at[pl.ds(tbase, TAIL)]],
                               rows_a.at[pl.ds(0, TAIL)], sem_a)
        cpt.wait()
        pltpu.sync_copy(rows_a.at[pl.ds(0, TAIL)],
                        out_hbm.at[k, pl.ds(qbase + tbase, TAIL)])


def _gather_sc(table, idx8):
    mesh = plsc.VectorSubcoreMesh(core_axis_name="c", subcore_axis_name="s")
    run = functools.partial(
        pl.kernel,
        out_type=jax.ShapeDtypeStruct((3, NQP, 32), jnp.float32),
        mesh=mesh,
        scratch_types=[
            pltpu.VMEM((Q_PER_W, 8), jnp.int32),
            pltpu.VMEM((Q_PER_W,), jnp.int32),
            pltpu.VMEM((Q_PER_W,), jnp.int32),
            pltpu.VMEM((Q_PER_W,), jnp.int32),
            pltpu.VMEM((CHUNK, 32), jnp.float32),
            pltpu.VMEM((CHUNK, 32), jnp.float32),
            pltpu.SemaphoreType.DMA,
            pltpu.SemaphoreType.DMA,
        ],
        compiler_params=pltpu.CompilerParams(use_tc_tiling_on_sc=False,
                                             needs_layout_passes=False),
    )(_gather_body)
    return run(table, idx8)


def _silu(x):
    return x / (1.0 + jnp.exp(-x))


def _dense_body(mq_ref, qu_ref, g_ref,
                meW1_ref, meb1_ref, meW2_ref, meb2_ref,
                peW1_ref, peb1_ref, peW2_ref, peb2_ref,
                mdW1a_ref, mdW1b_ref, mdb1_ref, mdW2_ref, mdb2_ref,
                mdW3_ref, mdb3_ref,
                ldW1a_ref, ldW1b_ref, ldb1_ref, ldW2_ref, ldb2_ref,
                ldW3_ref, ldb3_ref,
                om_ref, oq_ref, of_ref):
    means3 = mq_ref[...]
    q4 = qu_ref[...]
    rows = means3.shape[0]
    mq = jnp.concatenate(
        [means3, q4, jnp.ones((rows, 1), jnp.float32)], axis=1)
    g0 = g_ref[0]
    g1 = g_ref[1]
    g2 = g_ref[2]
    v0 = g0[:, 24:27]
    e1 = g1[:, 24:27] - v0
    e2 = g2[:, 24:27] - v0
    ep = means3 - v0
    d00 = jnp.sum(e1 * e1, axis=1, keepdims=True)
    d01 = jnp.sum(e1 * e2, axis=1, keepdims=True)
    d11 = jnp.sum(e2 * e2, axis=1, keepdims=True)
    d20 = jnp.sum(ep * e1, axis=1, keepdims=True)
    d21 = jnp.sum(ep * e2, axis=1, keepdims=True)
    denom = d00 * d11 - d01 * d01 + 1e-8
    v = (d11 * d20 - d01 * d21) / denom
    w = (d00 * d21 - d01 * d20) / denom
    u = 1.0 - v - w

    nm = u * g0 + v * g1 + w * g2

    def mm(a, w_ref):
        return jnp.dot(a, w_ref[...], preferred_element_type=jnp.float32)

    x = _silu(mm(_silu(mm(nm, meW1_ref) + meb1_ref[...]), meW2_ref) + meb2_ref[...])
    pe = _silu(mm(_silu(mm(mq, peW1_ref) + peb1_ref[...]), peW2_ref) + peb2_ref[...])

    h = _silu(mm(x, mdW1a_ref) + mm(pe, mdW1b_ref) + mdb1_ref[...])
    h = _silu(mm(h, mdW2_ref) + mdb2_ref[...])
    mv = mm(h, mdW3_ref) + mdb3_ref[...]

    lane = lax.broadcasted_iota(jnp.int32, (rows, 8), 1)
    scale = jnp.where(lane < 3, jnp.float32(0.001),
                      jnp.where(lane < 7, jnp.float32(0.01), jnp.float32(0.0)))
    upd8 = mq + scale * mv

    l = _silu(mm(x, ldW1a_ref) + mm(pe, ldW1b_ref) + ldb1_ref[...])
    l = _silu(mm(l, ldW2_ref) + ldb2_ref[...])
    feat = mm(l, ldW3_ref) + ldb3_ref[...]

    om_ref[...] = upd8[:, 0:3]
    oq_ref[...] = upd8[:, 3:7]
    of_ref[...] = feat


def _dense(means, quats, g3, weights):
    blk = 1024
    w_specs = [pl.BlockSpec(w.shape, lambda i: tuple(0 for _ in w.shape))
               for w in weights]
    return pl.pallas_call(
        _dense_body,
        grid=(NQP // blk,),
        in_specs=[
            pl.BlockSpec((blk, 3), lambda i: (i, 0)),
            pl.BlockSpec((blk, 4), lambda i: (i, 0)),
            pl.BlockSpec((3, blk, 32), lambda i: (0, i, 0)),
        ] + w_specs,
        out_specs=[
            pl.BlockSpec((blk, 3), lambda i: (i, 0)),
            pl.BlockSpec((blk, 4), lambda i: (i, 0)),
            pl.BlockSpec((blk, 32), lambda i: (i, 0)),
        ],
        out_shape=[
            jax.ShapeDtypeStruct((N_GAUSS, 3), jnp.float32),
            jax.ShapeDtypeStruct((N_GAUSS, 4), jnp.float32),
            jax.ShapeDtypeStruct((N_GAUSS, LATENT), jnp.float32),
        ],
    )(means, quats, g3, *weights)


def _pad_rows(a, n):
    return jnp.concatenate(
        [a, jnp.zeros((n - a.shape[0],) + a.shape[1:], a.dtype)], axis=0)


def kernel(means, quats, features, flame_vertices, canonical_vertices,
           me_W1, me_b1, me_W2, me_b2, pe_W1, pe_b1, pe_W2, pe_b2,
           md_W1, md_b1, md_W2, md_b2, md_W3, md_b3,
           ld_W1, ld_b1, ld_W2, ld_b2, ld_W3, ld_b3):
    f32 = jnp.float32
    cn2 = jnp.sum(canonical_vertices * canonical_vertices, axis=1)
    top = jnp.concatenate(
        [-2.0 * canonical_vertices.T, jnp.zeros((4, N_VERTS), f32), cn2[None]],
        axis=0)
    padcols = jnp.concatenate(
        [jnp.zeros((7, NVP - N_VERTS), f32),
         jnp.full((1, NVP - N_VERTS), _BIG, f32)], axis=0)
    cvt8 = jnp.concatenate([top, padcols], axis=1)

    idx8 = _knn(means, cvt8)

    vm = jnp.transpose(flame_vertices, (1, 0, 2)).reshape(N_VERTS, WINDOW * 3)
    table = jnp.concatenate(
        [vm, canonical_vertices, jnp.zeros((N_VERTS, 5), f32)], axis=1)
    table = _pad_rows(table, NVP)

    g3 = _gather_sc(table, idx8)

    z8 = jnp.zeros((8, HID), f32)
    meW1p = jnp.concatenate([me_W1, z8], axis=0)
    peW1p = jnp.concatenate([pe_W1, jnp.zeros((1, HID), f32)], axis=0)
    mdW3p = jnp.concatenate([md_W3, jnp.zeros((HID, 1), f32)], axis=1)
    mdb3p = jnp.concatenate([md_b3, jnp.zeros((1,), f32)])[None]
    weights = [
        meW1p, me_b1[None], me_W2, me_b2[None],
        peW1p, pe_b1[None], pe_W2, pe_b2[None],
        md_W1[:HID], md_W1[HID:], md_b1[None], md_W2, md_b2[None],
        mdW3p, mdb3p,
        ld_W1[:HID], ld_W1[HID:], ld_b1[None], ld_W2, ld_b2[None],
        ld_W3, ld_b3[None],
    ]
    new_means, new_quats, new_features = _dense(means, quats, g3, weights)
    return (new_means, new_quats, new_features, jnp.float32(0.0))
